# baseline (speedup 1.0000x reference)
import functools

import jax
import jax.numpy as jnp
from jax import lax
from jax.experimental import pallas as pl
from jax.experimental.pallas import tpu as pltpu
from jax.experimental.pallas import tpu_sc as plsc

HS = 300000
D = 64
NC = 2
NS = 16
L = 16
NW = NC * NS
INV_HS = 1.0 / HS
SEQ = 200
CHUNK = 400
GW = 80
NG = CHUNK // GW
NSLOT = 4


def _mod_hs(x):
    q = (x.astype(jnp.float32) * INV_HS).astype(jnp.int32)
    r = x - q * HS
    r = jnp.where(r >= HS, r - HS, r)
    r = jnp.where(r < 0, r + HS, r)
    return r


@functools.cache
def _make_kernel(B):
    n_per_w = B // NW
    n_chunks = n_per_w // CHUNK
    assert n_chunks % NSLOT == 0 and CHUNK % SEQ == 0
    mesh = plsc.VectorSubcoreMesh(core_axis_name="c", subcore_axis_name="s")

    scratch = [pltpu.VMEM((CHUNK + L,), jnp.int32)]
    for _ in range(NSLOT):
        scratch += [pltpu.VMEM((CHUNK,), jnp.int32),
                    pltpu.VMEM((CHUNK,), jnp.int32),
                    pltpu.VMEM((CHUNK, D), jnp.float32)]
    scratch += [pltpu.SemaphoreType.DMA] * (3 * NSLOT)

    @functools.partial(
        pl.kernel,
        out_type=jax.ShapeDtypeStruct((B, D), jnp.float32),
        mesh=mesh,
        compiler_params=pltpu.CompilerParams(use_tc_tiling_on_sc=False),
        scratch_types=scratch,
    )
    def k(ids_hbm, uni_hbm, bi_hbm, out_hbm, idsp, *rest):
        bufs = rest[:3 * NSLOT]
        sems = rest[3 * NSLOT:]
        slots = tuple(
            (bufs[3 * p], bufs[3 * p + 1], bufs[3 * p + 2],
             sems[3 * p], sems[3 * p + 1], sems[3 * p + 2])
            for p in range(NSLOT))
        wid = lax.axis_index("s") * NC + lax.axis_index("c")
        base_w = wid * n_per_w
        iota16 = lax.iota(jnp.int32, 16)

        def compute_idx(j, uidx, bidx):
            base = base_w + j * CHUNK
            pltpu.sync_copy(ids_hbm.at[pl.ds(base, CHUNK)],
                            idsp.at[pl.ds(L, CHUNK)])
            for m in range(0, CHUNK, L):
                ids16 = idsp[pl.ds(m + L, L)]
                prev16 = idsp[pl.ds(m + L - 1, L)]
                lane = (-m) % SEQ
                if lane < L:
                    prev16 = jnp.where(iota16 == lane, 0, prev16)
                sl = pl.ds(m, L)
                uidx[sl] = _mod_hs(ids16)
                bidx[sl] = _mod_hs(_mod_hs(prev16) * 31 + ids16)

        def fire_uni(uidx, buf, sem):
            for t in range(NG):
                pltpu.async_copy(uni_hbm.at[uidx.at[pl.ds(t * GW, GW)]],
                                 buf.at[pl.ds(t * GW, GW)], sem)

        def fire_bi_add(bidx, buf, sem):
            return [pltpu.async_copy(bi_hbm.at[bidx.at[pl.ds(t * GW, GW)]],
                                     buf.at[pl.ds(t * GW, GW)], sem, add=True)
                    for t in range(NG)]

        def drain_gathers(buf, sem):
            pltpu.make_async_copy(uni_hbm.at[pl.ds(0, CHUNK)], buf, sem).wait()

        def drain_write(buf, sw):
            pltpu.make_async_copy(buf, out_hbm.at[pl.ds(0, CHUNK)], sw).wait()

        for jj in range(2):
            uidx, bidx, buf, su, sb, sw = slots[jj]
            compute_idx(jj, uidx, bidx)
            fire_uni(uidx, buf, su)

        @pl.loop(0, n_chunks, step=NSLOT)
        def _(j):
            for p in range(NSLOT):
                uidx, bidx, buf, su, sb, sw = slots[p]
                pn = (p + 2) % NSLOT
                uidx_n, bidx_n, buf_n, su_n, sb_n, sw_n = slots[pn]
                jj = j + p

                drain_gathers(buf, su)
                cps_b = fire_bi_add(bidx, buf, sb)

                @pl.when(jj + 2 < n_chunks)
                def _():
                    @pl.when(jj >= 2)
                    def _():
                        drain_write(buf_n, sw_n)
                    compute_idx(jj + 2, uidx_n, bidx_n)
                    fire_uni(uidx_n, buf_n, su_n)

                for cp in cps_b:
                    cp.wait()
                base = base_w + jj * CHUNK
                pltpu.async_copy(buf, out_hbm.at[pl.ds(base, CHUNK)], sw)

        for p in range(NSLOT):
            uidx, bidx, buf, su, sb, sw = slots[p]
            drain_write(buf, sw)

    return k


def kernel(input_ids, unigram_table, bigram_table):
    bt, s = input_ids.shape
    ids = input_ids.astype(jnp.int32)
    b = bt * s
    h = b // 2
    idsf = ids.reshape(b)
    k = _make_kernel(h)
    o1 = k(idsf[:h], unigram_table, bigram_table)
    o2 = k(idsf[h:], unigram_table, bigram_table)
    return jnp.concatenate([o1, o2], axis=0).reshape(bt, s, D)


# speedup vs baseline: 1.2160x; 1.2160x over previous
import functools

import jax
import jax.numpy as jnp
from jax import lax
from jax.experimental import pallas as pl
from jax.experimental.pallas import tpu as pltpu
from jax.experimental.pallas import tpu_sc as plsc

HS = 300000
D = 64
NC = 2
NS = 16
L = 16
NW = NC * NS
INV_HS = 1.0 / HS
SEQ = 200
CHUNK = 400
GW = 80
NG = CHUNK // GW
NSLOT = 4


def _mod_hs(x):
    q = (x.astype(jnp.float32) * INV_HS).astype(jnp.int32)
    r = x - q * HS
    r = jnp.where(r >= HS, r - HS, r)
    r = jnp.where(r < 0, r + HS, r)
    return r


@functools.cache
def _make_kernel(B):
    n_per_w = B // NW
    n_chunks = n_per_w // CHUNK
    assert n_chunks % NSLOT == 0 and CHUNK % SEQ == 0
    mesh = plsc.VectorSubcoreMesh(core_axis_name="c", subcore_axis_name="s")

    scratch = [pltpu.VMEM((CHUNK + L,), jnp.int32)]
    for _ in range(NSLOT):
        scratch += [pltpu.VMEM((CHUNK,), jnp.int32),
                    pltpu.VMEM((CHUNK,), jnp.int32),
                    pltpu.VMEM((CHUNK, D), jnp.float32)]
    scratch += [pltpu.SemaphoreType.DMA] * (3 * NSLOT)

    @functools.partial(
        pl.kernel,
        out_type=jax.ShapeDtypeStruct((B, D), jnp.float32),
        mesh=mesh,
        compiler_params=pltpu.CompilerParams(use_tc_tiling_on_sc=False),
        scratch_types=scratch,
    )
    def k(ids_hbm, uni_hbm, bi_hbm, out_hbm, idsp, *rest):
        bufs = rest[:3 * NSLOT]
        sems = rest[3 * NSLOT:]
        slots = tuple(
            (bufs[3 * p], bufs[3 * p + 1], bufs[3 * p + 2],
             sems[3 * p], sems[3 * p + 1], sems[3 * p + 2])
            for p in range(NSLOT))
        wid = lax.axis_index("s") * NC + lax.axis_index("c")
        base_w = wid * n_per_w
        iota16 = lax.iota(jnp.int32, 16)

        def compute_idx(j, uidx, bidx):
            base = base_w + j * CHUNK
            pltpu.sync_copy(ids_hbm.at[pl.ds(base, CHUNK)],
                            idsp.at[pl.ds(L, CHUNK)])
            for m in range(0, CHUNK, L):
                ids16 = idsp[pl.ds(m + L, L)]
                prev16 = idsp[pl.ds(m + L - 1, L)]
                lane = (-m) % SEQ
                if lane < L:
                    prev16 = jnp.where(iota16 == lane, 0, prev16)
                sl = pl.ds(m, L)
                uidx[sl] = _mod_hs(ids16)
                bidx[sl] = _mod_hs(_mod_hs(prev16) * 31 + ids16)

        def fire_uni(uidx, buf, sem):
            for t in range(NG):
                pltpu.async_copy(uni_hbm.at[uidx.at[pl.ds(t * GW, GW)]],
                                 buf.at[pl.ds(t * GW, GW)], sem)

        def fire_bi_add(bidx, buf, sem):
            return [pltpu.async_copy(bi_hbm.at[bidx.at[pl.ds(t * GW, GW)]],
                                     buf.at[pl.ds(t * GW, GW)], sem, add=True)
                    for t in range(NG)]

        def drain_gathers(buf, sem):
            pltpu.make_async_copy(uni_hbm.at[pl.ds(0, CHUNK)], buf, sem).wait()

        def drain_write(buf, sw):
            pltpu.make_async_copy(buf, out_hbm.at[pl.ds(0, CHUNK)], sw).wait()

        for jj in range(2):
            uidx, bidx, buf, su, sb, sw = slots[jj]
            compute_idx(jj, uidx, bidx)
            fire_uni(uidx, buf, su)

        @pl.loop(0, n_chunks, step=NSLOT)
        def _(j):
            for p in range(NSLOT):
                uidx, bidx, buf, su, sb, sw = slots[p]
                pn = (p + 2) % NSLOT
                uidx_n, bidx_n, buf_n, su_n, sb_n, sw_n = slots[pn]
                jj = j + p

                drain_gathers(buf, su)
                cps_b = fire_bi_add(bidx, buf, sb)

                @pl.when(jj + 2 < n_chunks)
                def _():
                    @pl.when(jj >= 2)
                    def _():
                        drain_write(buf_n, sw_n)
                    compute_idx(jj + 2, uidx_n, bidx_n)
                    fire_uni(uidx_n, buf_n, su_n)

                for cp in cps_b:
                    cp.wait()
                base = base_w + jj * CHUNK
                pltpu.async_copy(buf, out_hbm.at[pl.ds(base, CHUNK)], sw)

        for p in range(NSLOT):
            uidx, bidx, buf, su, sb, sw = slots[p]
            drain_write(buf, sw)

    return k


def kernel(input_ids, unigram_table, bigram_table):
    bt, s = input_ids.shape
    ids = input_ids.astype(jnp.int32)
    b = bt * s
    out = _make_kernel(b)(ids.reshape(b), unigram_table, bigram_table)
    return out.reshape(bt, s, D)


# speedup vs baseline: 1.2175x; 1.0012x over previous
import functools

import jax
import jax.numpy as jnp
from jax import lax
from jax.experimental import pallas as pl
from jax.experimental.pallas import tpu as pltpu
from jax.experimental.pallas import tpu_sc as plsc

HS = 300000
D = 64
NC = 2
NS = 16
L = 16
NW = NC * NS
INV_HS = 1.0 / HS
SEQ = 200
CHUNK = 400
GW = 80
NG = CHUNK // GW
NSLOT = 4


def _mod_hs(x):
    q = (x.astype(jnp.float32) * INV_HS).astype(jnp.int32)
    r = x - q * HS
    r = jnp.where(r >= HS, r - HS, r)
    r = jnp.where(r < 0, r + HS, r)
    return r


@functools.cache
def _make_kernel(B):
    n_per_w = B // NW
    n_chunks = n_per_w // CHUNK
    assert n_chunks % NSLOT == 0 and CHUNK % SEQ == 0
    mesh = plsc.VectorSubcoreMesh(core_axis_name="c", subcore_axis_name="s")

    scratch = [pltpu.VMEM((CHUNK + L,), jnp.int32)]
    for _ in range(NSLOT):
        scratch += [pltpu.VMEM((CHUNK,), jnp.int32),
                    pltpu.VMEM((CHUNK,), jnp.int32),
                    pltpu.VMEM((CHUNK, D), jnp.float32)]
    scratch += [pltpu.SemaphoreType.DMA] * (3 * NSLOT)

    @functools.partial(
        pl.kernel,
        out_type=jax.ShapeDtypeStruct((B, D), jnp.float32),
        mesh=mesh,
        compiler_params=pltpu.CompilerParams(use_tc_tiling_on_sc=False),
        scratch_types=scratch,
    )
    def k(ids_hbm, uni_hbm, bi_hbm, out_hbm, idsp, *rest):
        bufs = rest[:3 * NSLOT]
        sems = rest[3 * NSLOT:]
        slots = tuple(
            (bufs[3 * p], bufs[3 * p + 1], bufs[3 * p + 2],
             sems[3 * p], sems[3 * p + 1], sems[3 * p + 2])
            for p in range(NSLOT))
        wid = lax.axis_index("s") * NC + lax.axis_index("c")
        base_w = wid * n_per_w
        iota16 = lax.iota(jnp.int32, 16)

        def compute_idx(j, uidx, bidx):
            base = base_w + j * CHUNK
            pltpu.sync_copy(ids_hbm.at[pl.ds(base, CHUNK)],
                            idsp.at[pl.ds(L, CHUNK)])

            @pl.loop(0, CHUNK, step=L)
            def _(m):
                ids16 = idsp[pl.ds(m + L, L)]
                prev16 = idsp[pl.ds(m + L - 1, L)]
                p = iota16 + m
                p = jnp.where(p >= 2 * SEQ, p - 2 * SEQ, p)
                p = jnp.where(p >= SEQ, p - SEQ, p)
                prev16 = jnp.where(p == 0, 0, prev16)
                sl = pl.ds(m, L)
                uidx[sl] = _mod_hs(ids16)
                bidx[sl] = _mod_hs(_mod_hs(prev16) * 31 + ids16)

        def fire_uni(uidx, buf, sem):
            for t in range(NG):
                pltpu.async_copy(uni_hbm.at[uidx.at[pl.ds(t * GW, GW)]],
                                 buf.at[pl.ds(t * GW, GW)], sem)

        def fire_bi_add(bidx, buf, sem):
            return [pltpu.async_copy(bi_hbm.at[bidx.at[pl.ds(t * GW, GW)]],
                                     buf.at[pl.ds(t * GW, GW)], sem, add=True)
                    for t in range(NG)]

        def drain_gathers(buf, sem):
            pltpu.make_async_copy(uni_hbm.at[pl.ds(0, CHUNK)], buf, sem).wait()

        def drain_write(buf, sw):
            pltpu.make_async_copy(buf, out_hbm.at[pl.ds(0, CHUNK)], sw).wait()

        for jj in range(2):
            uidx, bidx, buf, su, sb, sw = slots[jj]
            compute_idx(jj, uidx, bidx)
            fire_uni(uidx, buf, su)

        @pl.loop(0, n_chunks, step=NSLOT)
        def _(j):
            for p in range(NSLOT):
                uidx, bidx, buf, su, sb, sw = slots[p]
                pn = (p + 2) % NSLOT
                uidx_n, bidx_n, buf_n, su_n, sb_n, sw_n = slots[pn]
                jj = j + p

                drain_gathers(buf, su)
                cps_b = fire_bi_add(bidx, buf, sb)

                @pl.when(jj + 2 < n_chunks)
                def _():
                    @pl.when(jj >= 2)
                    def _():
                        drain_write(buf_n, sw_n)
                    compute_idx(jj + 2, uidx_n, bidx_n)
                    fire_uni(uidx_n, buf_n, su_n)

                for cp in cps_b:
                    cp.wait()
                base = base_w + jj * CHUNK
                pltpu.async_copy(buf, out_hbm.at[pl.ds(base, CHUNK)], sw)

        for p in range(NSLOT):
            uidx, bidx, buf, su, sb, sw = slots[p]
            drain_write(buf, sw)

    return k


def kernel(input_ids, unigram_table, bigram_table):
    bt, s = input_ids.shape
    ids = input_ids.astype(jnp.int32)
    b = bt * s
    out = _make_kernel(b)(ids.reshape(b), unigram_table, bigram_table)
    return out.reshape(bt, s, D)
